# Initial kernel scaffold; baseline (speedup 1.0000x reference)
import jax
import jax.numpy as jnp
from jax.experimental import pallas as pl


def kernel(x, edge_index, edge_weight, W1, b1, W2, b2, Wfc, bfc):
    raise NotImplementedError("write your pallas kernel here")


# speedup vs baseline: 9.9653x; 9.9653x over previous
import functools

import jax
import jax.numpy as jnp
from jax import lax
from jax.experimental import pallas as pl
from jax.experimental.pallas import tpu as pltpu
from jax.experimental.pallas import tpu_sc as plsc

N = 10000
E = 320000
D = 128
NC = 2
NS = 16
NW = NC * NS
E_PER_TILE = E // NW
CHUNK = 80
NCHUNK = E_PER_TILE // CHUNK
NPAD = 10240
SEG = NPAD // NS
ZROWS = 128

_mesh = plsc.VectorSubcoreMesh(
    core_axis_name="c", subcore_axis_name="s", num_cores=NC, num_subcores=NS)


@functools.partial(
    pl.kernel,
    out_type=jax.ShapeDtypeStruct((NC, NPAD), jnp.float32),
    mesh=_mesh,
    scratch_types=[
        pltpu.VMEM((CHUNK,), jnp.int32),
        pltpu.VMEM((CHUNK,), jnp.float32),
        pltpu.VMEM((SEG,), jnp.float32),
        pltpu.VMEM_SHARED((NPAD,), jnp.float32),
    ],
)
def _deg_kernel(dst_hbm, w_hbm, out_hbm, dstv, wv, zv, acc):
    c = lax.axis_index("c")
    s = lax.axis_index("s")
    tile = c * NS + s
    for j in range(SEG // 16):
        zv[pl.ds(j * 16, 16)] = jnp.zeros((16,), jnp.float32)
    pltpu.sync_copy(zv, acc.at[pl.ds(s * SEG, SEG)])
    plsc.subcore_barrier()

    base = tile * E_PER_TILE

    def body(i, carry):
        off = base + i * CHUNK
        pltpu.sync_copy(dst_hbm.at[pl.ds(off, CHUNK)], dstv)
        pltpu.sync_copy(w_hbm.at[pl.ds(off, CHUNK)], wv)
        pltpu.sync_copy(wv, acc.at[dstv], add=True)
        return carry

    lax.fori_loop(0, NCHUNK, body, 0)
    plsc.subcore_barrier()
    pltpu.sync_copy(acc.at[pl.ds(s * SEG, SEG)], out_hbm.at[c, pl.ds(s * SEG, SEG)])


@functools.partial(
    pl.kernel,
    out_type=jax.ShapeDtypeStruct((NC, NPAD, D), jnp.float32),
    mesh=_mesh,
    scratch_types=[
        pltpu.VMEM((CHUNK,), jnp.int32),
        pltpu.VMEM((CHUNK,), jnp.int32),
        pltpu.VMEM((CHUNK,), jnp.float32),
        pltpu.VMEM((CHUNK, D), jnp.float32),
        pltpu.VMEM((ZROWS, D), jnp.float32),
        pltpu.VMEM_SHARED((NPAD, D), jnp.float32),
        pltpu.SemaphoreType.DMA,
    ],
)
def _edge_kernel(y_hbm, src_hbm, dst_hbm, w_hbm, out_hbm,
                 srcv, dstv, wv, rows, zbuf, acc, sem):
    c = lax.axis_index("c")
    s = lax.axis_index("s")
    tile = c * NS + s

    def zrow(i, carry):
        for k in range(D // 16):
            zbuf[i, pl.ds(k * 16, 16)] = jnp.zeros((16,), jnp.float32)
        return carry
    lax.fori_loop(0, ZROWS, zrow, 0)
    for r in range(SEG // ZROWS):
        pltpu.sync_copy(zbuf, acc.at[pl.ds(s * SEG + r * ZROWS, ZROWS)])
    plsc.subcore_barrier()

    base = tile * E_PER_TILE

    def body(i, carry):
        off = base + i * CHUNK
        pltpu.sync_copy(src_hbm.at[pl.ds(off, CHUNK)], srcv)
        pltpu.sync_copy(dst_hbm.at[pl.ds(off, CHUNK)], dstv)
        pltpu.sync_copy(w_hbm.at[pl.ds(off, CHUNK)], wv)
        pltpu.async_copy(y_hbm.at[srcv], rows, sem).wait()

        def scale(g, carry2):
            e0 = g * 16
            wvec = wv[pl.ds(e0, 16)]
            for j in range(16):
                we = wvec[j]
                for k in range(D // 16):
                    rows[e0 + j, pl.ds(k * 16, 16)] = (
                        rows[e0 + j, pl.ds(k * 16, 16)] * we)
            return carry2
        lax.fori_loop(0, CHUNK // 16, scale, 0)

        pltpu.sync_copy(rows, acc.at[dstv], add=True)
        return carry

    lax.fori_loop(0, NCHUNK, body, 0)
    plsc.subcore_barrier()
    pltpu.sync_copy(acc.at[pl.ds(s * SEG, SEG)],
                    out_hbm.at[c, pl.ds(s * SEG, SEG)])


def _tc1_body(p0, p1, x, w1, dis_out, y_out):
    deg = 1.0 + p0[...] + p1[...]
    dis = lax.rsqrt(deg)
    dis_out[...] = dis
    xw = jnp.dot(x[...], w1[...], preferred_element_type=jnp.float32)
    y_out[...] = xw * dis


_tc1 = pl.pallas_call(
    _tc1_body,
    out_shape=[jax.ShapeDtypeStruct((N, 1), jnp.float32),
               jax.ShapeDtypeStruct((N, D), jnp.float32)],
)


def _tc2_body(dis, a0, a1, y, b, w2, y2_out):
    h = jnp.maximum(dis[...] * (a0[...] + a1[...] + y[...]) + b[...], 0.0)
    y2_out[...] = jnp.dot(h, w2[...], preferred_element_type=jnp.float32) * dis[...]


_tc2 = pl.pallas_call(
    _tc2_body,
    out_shape=jax.ShapeDtypeStruct((N, D), jnp.float32),
)


def _tc3_body(dis, a0, a1, y, b, wfc, bfc, out):
    h = jnp.maximum(dis[...] * (a0[...] + a1[...] + y[...]) + b[...], 0.0)
    z = jnp.dot(h, wfc[...], preferred_element_type=jnp.float32) + bfc[...]
    out[...] = jax.nn.sigmoid(z)


_tc3 = pl.pallas_call(
    _tc3_body,
    out_shape=jax.ShapeDtypeStruct((N, 1), jnp.float32),
)


def kernel(x, edge_index, edge_weight, W1, b1, W2, b2, Wfc, bfc):
    src = edge_index[0]
    dst = edge_index[1]
    degp = _deg_kernel(dst, edge_weight)
    p0 = degp[0, :N, None]
    p1 = degp[1, :N, None]
    dis, y1 = _tc1(p0, p1, x, W1)
    acc1 = _edge_kernel(y1, src, dst, edge_weight)
    y2 = _tc2(dis, acc1[0, :N], acc1[1, :N], y1, b1.reshape(1, D), W2)
    acc2 = _edge_kernel(y2, src, dst, edge_weight)
    out = _tc3(dis, acc2[0, :N], acc2[1, :N], y2, b2.reshape(1, D),
               Wfc, bfc.reshape(1, 1))
    return out.reshape(-1)
